# initial kernel scaffold (unmeasured)
import jax
import jax.numpy as jnp
from jax import lax
from jax.experimental import pallas as pl
from jax.experimental.pallas import tpu as pltpu

N_DEV = 4
M = 4096
K_LOC = 1024
M_LOC = 1024
N = 2048

FP8 = jnp.float8_e4m3fn


def kernel(x, w_mat, scale_x, scale_w):

    def body(x_ref, w_hbm, sx_ref, sw_ref, out_ref,
             x8_ref, xfull_ref, wstage_ref, w8_ref,
             local_sem, copy_sem, send_sems, recv_sems):
        my = lax.axis_index("i")

        x8_ref[...] = x_ref[...].astype(FP8)

        barrier = pltpu.get_barrier_semaphore()
        for d in (1, 2, 3):
            peer = lax.rem(my + d, N_DEV)
            pl.semaphore_signal(
                barrier, inc=1,
                device_id=(peer,), device_id_type=pl.DeviceIdType.MESH,
            )
        pl.semaphore_wait(barrier, 3)

        rdmas = []
        for d in (1, 2, 3):
            peer = lax.rem(my + d, N_DEV)
            rdma = pltpu.make_async_remote_copy(
                src_ref=x8_ref.at[pl.ds(peer * M_LOC, M_LOC), :],
                dst_ref=xfull_ref.at[my],
                send_sem=send_sems.at[d - 1],
                recv_sem=recv_sems.at[d - 1],
                device_id=(peer,),
                device_id_type=pl.DeviceIdType.MESH,
            )
            rdma.start()
            rdmas.append(rdma)

        local_cp = pltpu.make_async_copy(
            x8_ref.at[pl.ds(my * M_LOC, M_LOC), :],
            xfull_ref.at[my],
            local_sem,
        )
        local_cp.start()

        for t in range(N_DEV):
            cp = pltpu.make_async_copy(
                w_hbm.at[pl.ds(t * K_LOC, K_LOC), :], wstage_ref, copy_sem,
            )
            cp.start()
            cp.wait()
            w8_ref[t] = wstage_ref[...].astype(FP8)

        local_cp.wait()
        for rdma in rdmas:
            rdma.wait()

        out_ref[...] = jnp.dot(
            xfull_ref[0], w8_ref[0], preferred_element_type=jnp.float32
        )
        for s in range(1, N_DEV):
            out_ref[...] += jnp.dot(
                xfull_ref[s], w8_ref[s], preferred_element_type=jnp.float32
            )

        scale = sx_ref[0] * sw_ref[0]
        out_ref[...] = jnp.maximum(out_ref[...] * scale, 0.0)

    return pl.pallas_call(
        body,
        out_shape=jax.ShapeDtypeStruct((M_LOC, N), jnp.float32),
        in_specs=[
            pl.BlockSpec(memory_space=pltpu.VMEM),
            pl.BlockSpec(memory_space=pltpu.ANY),
            pl.BlockSpec(memory_space=pltpu.SMEM),
            pl.BlockSpec(memory_space=pltpu.SMEM),
        ],
        out_specs=pl.BlockSpec(memory_space=pltpu.VMEM),
        scratch_shapes=[
            pltpu.VMEM((M, K_LOC), FP8),
            pltpu.VMEM((N_DEV, M_LOC, K_LOC), FP8),
            pltpu.VMEM((K_LOC, N), jnp.float32),
            pltpu.VMEM((N_DEV, K_LOC, N), FP8),
            pltpu.SemaphoreType.DMA,
            pltpu.SemaphoreType.DMA,
            pltpu.SemaphoreType.DMA((3,)),
            pltpu.SemaphoreType.DMA((3,)),
        ],
        compiler_params=pltpu.CompilerParams(collective_id=0),
    )(x, w_mat, scale_x, scale_w)


# baseline (device time: 53728 ns/iter reference)
import jax
import jax.numpy as jnp
from jax import lax
from jax.experimental import pallas as pl
from jax.experimental.pallas import tpu as pltpu

N_DEV = 4
M = 4096
K_LOC = 1024
M_LOC = 1024
N = 2048

FP8 = jnp.float8_e4m3fn


def kernel(x, w_mat, scale_x, scale_w):

    def body(x_ref, w_hbm, sx_ref, sw_ref, out_ref,
             x8_ref, xfull_ref, wstage_ref, w8_ref,
             local_sem, copy_sem, send_sems, recv_sems):
        my = lax.axis_index("i")

        x8_ref[...] = x_ref[...].astype(FP8)

        barrier = pltpu.get_barrier_semaphore()
        for d in (1, 2, 3):
            peer = lax.rem(my + d, N_DEV)
            pl.semaphore_signal(
                barrier, inc=1,
                device_id=(peer,), device_id_type=pl.DeviceIdType.MESH,
            )
        pl.semaphore_wait(barrier, 3)

        rdmas = []
        for d in (1, 2, 3):
            peer = lax.rem(my + d, N_DEV)
            rdma = pltpu.make_async_remote_copy(
                src_ref=x8_ref.at[pl.ds(peer * M_LOC, M_LOC), :],
                dst_ref=xfull_ref.at[my],
                send_sem=send_sems.at[d - 1],
                recv_sem=recv_sems.at[d - 1],
                device_id=(peer,),
                device_id_type=pl.DeviceIdType.MESH,
            )
            rdma.start()
            rdmas.append(rdma)

        local_cp = pltpu.make_async_copy(
            x8_ref.at[pl.ds(my * M_LOC, M_LOC), :],
            xfull_ref.at[my],
            local_sem,
        )
        local_cp.start()

        for t in range(N_DEV):
            cp = pltpu.make_async_copy(
                w_hbm.at[pl.ds(t * K_LOC, K_LOC), :], wstage_ref, copy_sem,
            )
            cp.start()
            cp.wait()
            w8_ref[t] = wstage_ref[...].astype(FP8)

        local_cp.wait()
        for rdma in rdmas:
            rdma.wait()

        out_ref[...] = jnp.dot(
            xfull_ref[0], w8_ref[0], preferred_element_type=jnp.float32
        )
        for s in range(1, N_DEV):
            out_ref[...] += jnp.dot(
                xfull_ref[s], w8_ref[s], preferred_element_type=jnp.float32
            )

        scale = sx_ref[0] * sw_ref[0]
        out_ref[...] = jnp.maximum(out_ref[...] * scale, 0.0)

    return pl.pallas_call(
        body,
        out_shape=jax.ShapeDtypeStruct((M_LOC, N), jnp.float32),
        in_specs=[
            pl.BlockSpec(memory_space=pltpu.VMEM),
            pl.BlockSpec(memory_space=pl.ANY),
            pl.BlockSpec(memory_space=pltpu.SMEM),
            pl.BlockSpec(memory_space=pltpu.SMEM),
        ],
        out_specs=pl.BlockSpec(memory_space=pltpu.VMEM),
        scratch_shapes=[
            pltpu.VMEM((M, K_LOC), FP8),
            pltpu.VMEM((N_DEV, M_LOC, K_LOC), FP8),
            pltpu.VMEM((K_LOC, N), jnp.float32),
            pltpu.VMEM((N_DEV, K_LOC, N), FP8),
            pltpu.SemaphoreType.DMA,
            pltpu.SemaphoreType.DMA,
            pltpu.SemaphoreType.DMA((3,)),
            pltpu.SemaphoreType.DMA((3,)),
        ],
        compiler_params=pltpu.CompilerParams(
            collective_id=0,
            vmem_limit_bytes=60 * 1024 * 1024,
        ),
    )(x, w_mat, scale_x, scale_w)


# device time: 43530 ns/iter; 1.2343x vs baseline; 1.2343x over previous
import jax
import jax.numpy as jnp
from jax import lax
from jax.experimental import pallas as pl
from jax.experimental.pallas import tpu as pltpu

N_DEV = 4
M = 4096
K_LOC = 1024
M_LOC = 1024
N = 2048

FP8 = jnp.float8_e4m3fn

SEND_ORDER = (1, 3, 2)


def kernel(x, w_mat, scale_x, scale_w):

    def body(x_hbm, w_hbm, sx_ref, sw_ref, out_ref,
             xstage_ref, x8s_ref, xfull_ref, wstage_ref, w8_ref,
             xload_sems, wcopy_sem, send_sems, recv_sems):
        my = lax.axis_index("i")

        xloads = []
        for o, d in enumerate(SEND_ORDER):
            peer = lax.rem(my + d, N_DEV)
            cp = pltpu.make_async_copy(
                x_hbm.at[pl.ds(peer * M_LOC, M_LOC), :],
                xstage_ref.at[o],
                xload_sems.at[o],
            )
            cp.start()
            xloads.append(cp)
        cp_local = pltpu.make_async_copy(
            x_hbm.at[pl.ds(my * M_LOC, M_LOC), :],
            xstage_ref.at[3],
            xload_sems.at[3],
        )
        cp_local.start()

        barrier = pltpu.get_barrier_semaphore()
        for d in (1, 2, 3):
            peer = lax.rem(my + d, N_DEV)
            pl.semaphore_signal(
                barrier, inc=1,
                device_id=(peer,), device_id_type=pl.DeviceIdType.MESH,
            )
        pl.semaphore_wait(barrier, 3)

        rdmas = {}
        for o, d in enumerate(SEND_ORDER):
            peer = lax.rem(my + d, N_DEV)
            xloads[o].wait()
            x8s_ref[o] = xstage_ref[o].astype(FP8)
            rdma = pltpu.make_async_remote_copy(
                src_ref=x8s_ref.at[o],
                dst_ref=xfull_ref.at[d],
                send_sem=send_sems.at[d - 1],
                recv_sem=recv_sems.at[d - 1],
                device_id=(peer,),
                device_id_type=pl.DeviceIdType.MESH,
            )
            rdma.start()
            rdmas[d] = rdma

        cp_local.wait()
        xfull_ref[0] = xstage_ref[3].astype(FP8)

        for d in (0, 1, 3, 2):
            src = lax.rem(my - d + N_DEV, N_DEV)
            cp = pltpu.make_async_copy(
                w_hbm.at[pl.ds(src * K_LOC, K_LOC), :], wstage_ref, wcopy_sem,
            )
            cp.start()
            cp.wait()
            w8_ref[d] = wstage_ref[...].astype(FP8)

        out_ref[...] = jnp.dot(
            xfull_ref[0], w8_ref[0], preferred_element_type=jnp.float32
        )
        for d in (1, 3, 2):
            rdmas[d].wait_recv()
            out_ref[...] += jnp.dot(
                xfull_ref[d], w8_ref[d], preferred_element_type=jnp.float32
            )
        scale = sx_ref[0] * sw_ref[0]
        out_ref[...] = jnp.maximum(out_ref[...] * scale, 0.0)

        for d in (1, 3, 2):
            rdmas[d].wait_send()

    return pl.pallas_call(
        body,
        out_shape=jax.ShapeDtypeStruct((M_LOC, N), jnp.float32),
        in_specs=[
            pl.BlockSpec(memory_space=pl.ANY),
            pl.BlockSpec(memory_space=pl.ANY),
            pl.BlockSpec(memory_space=pltpu.SMEM),
            pl.BlockSpec(memory_space=pltpu.SMEM),
        ],
        out_specs=pl.BlockSpec(memory_space=pltpu.VMEM),
        scratch_shapes=[
            pltpu.VMEM((N_DEV, M_LOC, K_LOC), jnp.float32),
            pltpu.VMEM((3, M_LOC, K_LOC), FP8),
            pltpu.VMEM((N_DEV, M_LOC, K_LOC), FP8),
            pltpu.VMEM((K_LOC, N), jnp.float32),
            pltpu.VMEM((N_DEV, K_LOC, N), FP8),
            pltpu.SemaphoreType.DMA((4,)),
            pltpu.SemaphoreType.DMA,
            pltpu.SemaphoreType.DMA((3,)),
            pltpu.SemaphoreType.DMA((3,)),
        ],
        compiler_params=pltpu.CompilerParams(
            collective_id=0,
            vmem_limit_bytes=60 * 1024 * 1024,
        ),
    )(x, w_mat, scale_x, scale_w)
